# baseline (speedup 1.0000x reference)
import functools

import jax
import jax.numpy as jnp
from jax.experimental import pallas as pl
from jax.experimental.pallas import tpu as pltpu


def _round_up(x: int, m: int) -> int:
    return ((x + m - 1) // m) * m


def _ffn_body(x_ref, w1_ref, b1_ref, w2_ref, b2_ref, o_ref, *, nsub):
    tm = x_ref.shape[0]
    rm = tm // nsub
    w1 = w1_ref[...]
    w2 = w2_ref[...]
    b1 = b1_ref[...]
    for j in range(nsub):
        xj = x_ref[j * rm:(j + 1) * rm, :].astype(w1.dtype)
        hj = jnp.dot(xj, w1, preferred_element_type=jnp.float32)
        hj = jnp.maximum(hj + b1, 0.0).astype(w2.dtype)
        yj = jnp.dot(hj, w2, preferred_element_type=jnp.float32)
        o_ref[j * rm:(j + 1) * rm, :] = yj + b2_ref[...]


@functools.partial(jax.jit, static_argnames=("tm", "nsub", "wdtype"))
def _ffn(x, w1, b1, w2, b2, tm=1024, nsub=2, wdtype=jnp.bfloat16):
    orig_shape = x.shape
    D = orig_shape[-1]
    F = w1.shape[1]
    x2d = x.reshape(-1, D)
    N = x2d.shape[0]

    Dp = _round_up(D, 128)
    Fp = _round_up(F, 128)
    tm = min(tm, _round_up(N, 8))
    Np = _round_up(N, tm)

    w1_b = w1.astype(wdtype)
    w2_b = w2.astype(wdtype)

    if (Np, Dp, Fp) == (N, D, F):
        x_p, w1_p, w2_p = x2d, w1_b, w2_b
        b1_p = b1.reshape(1, F)
        b2_p = b2.reshape(1, D)
    else:
        x_p = jnp.zeros((Np, Dp), x.dtype).at[:N, :D].set(x2d)
        w1_p = jnp.zeros((Dp, Fp), wdtype).at[:D, :F].set(w1_b)
        b1_p = jnp.zeros((1, Fp), b1.dtype).at[0, :F].set(b1)
        w2_p = jnp.zeros((Fp, Dp), wdtype).at[:F, :D].set(w2_b)
        b2_p = jnp.zeros((1, Dp), b2.dtype).at[0, :D].set(b2)

    out = pl.pallas_call(
        functools.partial(_ffn_body, nsub=nsub),
        out_shape=jax.ShapeDtypeStruct((Np, Dp), jnp.float32),
        grid=(Np // tm,),
        in_specs=[
            pl.BlockSpec((tm, Dp), lambda i: (i, 0)),
            pl.BlockSpec((Dp, Fp), lambda i: (0, 0)),
            pl.BlockSpec((1, Fp), lambda i: (0, 0)),
            pl.BlockSpec((Fp, Dp), lambda i: (0, 0)),
            pl.BlockSpec((1, Dp), lambda i: (0, 0)),
        ],
        out_specs=pl.BlockSpec((tm, Dp), lambda i: (i, 0)),
        compiler_params=pltpu.CompilerParams(
            dimension_semantics=("parallel",),
            vmem_limit_bytes=56 * 1024 * 1024,
        ),
        cost_estimate=pl.CostEstimate(
            flops=4 * Np * Dp * Fp,
            transcendentals=0,
            bytes_accessed=(2 * Np * Dp + Fp + Dp) * 4 + 2 * Dp * Fp * 2,
        ),
    )(x_p, w1_p, b1_p, w2_p, b2_p)

    if (Np, Dp) != (N, D):
        out = out[:N, :D]
    return out.reshape(orig_shape)


def kernel(x, w1, b1, w2, b2):
    return _ffn(x, w1, b1, w2, b2, tm=4096, nsub=4, wdtype=jnp.float32)


# speedup vs baseline: 1.0210x; 1.0210x over previous
import functools

import jax
import jax.numpy as jnp
from jax.experimental import pallas as pl
from jax.experimental.pallas import tpu as pltpu


def _round_up(x: int, m: int) -> int:
    return ((x + m - 1) // m) * m


def _ffn_body(x_ref, w1_ref, b1_ref, w2_ref, b2_ref, o_ref, *, nsub):
    tm = x_ref.shape[0]
    rm = tm // nsub
    w1 = w1_ref[...]
    w2 = w2_ref[...]
    b1 = b1_ref[...]
    for j in range(nsub):
        xj = x_ref[j * rm:(j + 1) * rm, :].astype(w1.dtype)
        hj = jnp.dot(xj, w1, preferred_element_type=jnp.float32)
        hj = jnp.maximum(hj + b1, 0.0).astype(w2.dtype)
        yj = jnp.dot(hj, w2, preferred_element_type=jnp.float32)
        o_ref[j * rm:(j + 1) * rm, :] = yj + b2_ref[...]


@functools.partial(jax.jit, static_argnames=("tm", "nsub", "wdtype"))
def _ffn(x, w1, b1, w2, b2, tm=1024, nsub=2, wdtype=jnp.bfloat16):
    orig_shape = x.shape
    D = orig_shape[-1]
    F = w1.shape[1]
    x2d = x.reshape(-1, D)
    N = x2d.shape[0]

    Dp = _round_up(D, 128)
    Fp = _round_up(F, 128)
    tm = min(tm, _round_up(N, 8))
    Np = _round_up(N, tm)

    w1_b = w1.astype(wdtype)
    w2_b = w2.astype(wdtype)

    if (Np, Dp, Fp) == (N, D, F):
        x_p, w1_p, w2_p = x2d, w1_b, w2_b
        b1_p = b1.reshape(1, F)
        b2_p = b2.reshape(1, D)
    else:
        x_p = jnp.zeros((Np, Dp), x.dtype).at[:N, :D].set(x2d)
        w1_p = jnp.zeros((Dp, Fp), wdtype).at[:D, :F].set(w1_b)
        b1_p = jnp.zeros((1, Fp), b1.dtype).at[0, :F].set(b1)
        w2_p = jnp.zeros((Fp, Dp), wdtype).at[:F, :D].set(w2_b)
        b2_p = jnp.zeros((1, Dp), b2.dtype).at[0, :D].set(b2)

    out = pl.pallas_call(
        functools.partial(_ffn_body, nsub=nsub),
        out_shape=jax.ShapeDtypeStruct((Np, Dp), jnp.float32),
        grid=(Np // tm,),
        in_specs=[
            pl.BlockSpec((tm, Dp), lambda i: (i, 0)),
            pl.BlockSpec((Dp, Fp), lambda i: (0, 0)),
            pl.BlockSpec((1, Fp), lambda i: (0, 0)),
            pl.BlockSpec((Fp, Dp), lambda i: (0, 0)),
            pl.BlockSpec((1, Dp), lambda i: (0, 0)),
        ],
        out_specs=pl.BlockSpec((tm, Dp), lambda i: (i, 0)),
        compiler_params=pltpu.CompilerParams(
            dimension_semantics=("parallel",),
            vmem_limit_bytes=56 * 1024 * 1024,
        ),
        cost_estimate=pl.CostEstimate(
            flops=4 * Np * Dp * Fp,
            transcendentals=0,
            bytes_accessed=(2 * Np * Dp + Fp + Dp) * 4 + 2 * Dp * Fp * 2,
        ),
    )(x_p, w1_p, b1_p, w2_p, b2_p)

    if (Np, Dp) != (N, D):
        out = out[:N, :D]
    return out.reshape(orig_shape)


def kernel(x, w1, b1, w2, b2):
    return _ffn(x, w1, b1, w2, b2, tm=2048, nsub=2, wdtype=jnp.float32)
